# baseline (speedup 1.0000x reference)
import functools

import jax
import jax.numpy as jnp
from jax import lax
from jax.experimental import pallas as pl
from jax.experimental.pallas import tpu as pltpu
from jax.experimental.pallas import tpu_sc as plsc

NC = 2
NS = 16
NW = NC * NS
L = 16

C = 16


def _sc_body(batch, seq_len, embed, x_hbm, pos_hbm, out_hbm, pos_v, x_v):
    wid = lax.axis_index("s") * NC + lax.axis_index("c")
    seq_per_w = seq_len // NW
    n_chunks = seq_per_w // C
    vecs_per_chunk = (C * embed) // L

    def chunk_body(ci, _):
        s0 = wid * seq_per_w + ci * C
        pltpu.sync_copy(pos_hbm.at[pl.ds(s0, C), :], pos_v)
        for b in range(batch):
            pltpu.sync_copy(x_hbm.at[b, pl.ds(s0, C), :], x_v.at[b])

        def add_body(i, _):
            r = i // (embed // L)
            j = (i % (embed // L)) * L
            vp = pos_v[r, pl.ds(j, L)]
            for b in range(batch):
                x_v[b, r, pl.ds(j, L)] = x_v[b, r, pl.ds(j, L)] + vp
            return 0

        lax.fori_loop(0, vecs_per_chunk, add_body, 0)
        for b in range(batch):
            pltpu.sync_copy(x_v.at[b], out_hbm.at[b, pl.ds(s0, C), :])
        return 0

    lax.fori_loop(0, n_chunks, chunk_body, 0)


def kernel(x, position_embeddings):
    batch, seq_len, embed = x.shape
    pos = position_embeddings[:seq_len]
    mesh = plsc.VectorSubcoreMesh(core_axis_name="c", subcore_axis_name="s")
    body = functools.partial(_sc_body, batch, seq_len, embed)
    return pl.kernel(
        body,
        out_type=jax.ShapeDtypeStruct((batch, seq_len, embed), x.dtype),
        mesh=mesh,
        scratch_types=[
            pltpu.VMEM((C, embed), jnp.float32),
            pltpu.VMEM((batch, C, embed), jnp.float32),
        ],
    )(x, pos)
